# baseline (speedup 1.0000x reference)
import functools

import jax
import jax.numpy as jnp
from jax import lax
from jax.experimental import pallas as pl
from jax.experimental.pallas import tpu as pltpu
from jax.experimental.pallas import tpu_sc as plsc

B = 4
N_ANG = 180
N_DET = 384
SIN_SZ = N_ANG * N_DET
OUT_SZ = 256
NPIX = OUT_SZ * OUT_SZ
FILT_LEN = 129
HALF = (FILT_LEN - 1) // 2
PADDED = N_DET + 2 * HALF
NNZ = NPIX * N_ANG

NC = 2
NS = 16
NW = NC * NS
LANES = 16

PIX_PER_W = NPIX // NW
CHUNK_PIX = 128
NCHUNK = PIX_PER_W // CHUNK_PIX
CHUNK_NNZ = CHUNK_PIX * N_ANG
UNROLL = 6

ROW_MASK = (1 << 17) - 1
VAL_BIAS = 114688
HI_MASK = jnp.int32(-65536)


def _conv_pack_body(xp_ref, band_ref, p01_ref, p23_ref):
    band = band_ref[...]
    y = [jnp.dot(xp_ref[b], band, preferred_element_type=jnp.float32)
         for b in range(B)]
    r = [lax.bitcast_convert_type(v, jnp.uint32) + jnp.uint32(0x8000)
         for v in y]
    p01_ref[...] = lax.bitcast_convert_type(
        (r[0] >> 16) | (r[1] & jnp.uint32(0xFFFF0000)), jnp.int32)
    p23_ref[...] = lax.bitcast_convert_type(
        (r[2] >> 16) | (r[3] & jnp.uint32(0xFFFF0000)), jnp.int32)


_conv_pack = pl.pallas_call(
    _conv_pack_body,
    out_shape=[jax.ShapeDtypeStruct((N_ANG, N_DET), jnp.int32),
               jax.ShapeDtypeStruct((N_ANG, N_DET), jnp.int32)],
)

PK_COLS = 1024
PK_ROWS = NNZ // PK_COLS
PK_BLK = 240


def _pack_nnz_body(rows_ref, vals_ref, pk_ref):
    vbits = lax.bitcast_convert_type(vals_ref[...], jnp.uint32)
    vb = ((vbits + jnp.uint32(0x1000)) >> 13).astype(jnp.int32) - VAL_BIAS
    vb = jnp.clip(vb, 0, (1 << 15) - 1)
    pk_ref[...] = rows_ref[...] | (vb << 17)


_pack_nnz = pl.pallas_call(
    _pack_nnz_body,
    grid=(PK_ROWS // PK_BLK,),
    in_specs=[pl.BlockSpec((PK_BLK, PK_COLS), lambda i: (i, 0)),
              pl.BlockSpec((PK_BLK, PK_COLS), lambda i: (i, 0))],
    out_specs=pl.BlockSpec((PK_BLK, PK_COLS), lambda i: (i, 0)),
    out_shape=jax.ShapeDtypeStruct((PK_ROWS, PK_COLS), jnp.int32),
)


@functools.lru_cache(maxsize=1)
def _make_backproject():
    mesh = plsc.VectorSubcoreMesh(
        core_axis_name="c", subcore_axis_name="s",
        num_cores=NC, num_subcores=NS)

    @functools.partial(
        pl.kernel,
        mesh=mesh,
        compiler_params=pltpu.CompilerParams(needs_layout_passes=False),
        out_type=jax.ShapeDtypeStruct((B, NPIX), jnp.float32),
        scratch_types=[
            pltpu.VMEM((SIN_SZ,), jnp.int32),
            pltpu.VMEM((CHUNK_NNZ,), jnp.int32),
            pltpu.VMEM((CHUNK_NNZ,), jnp.int32),
            pltpu.VMEM((PIX_PER_W,), jnp.float32),
            pltpu.VMEM((PIX_PER_W,), jnp.float32),
            pltpu.VMEM((LANES,), jnp.float32),
            pltpu.SemaphoreType.DMA((2,)),
        ],
    )
    def backproject(ptab_hbm, pk_hbm, bias_hbm, out_hbm,
                    table_v, pk0_v, pk1_v, out0_v, out1_v, bias_v, sems):
        wid = lax.axis_index("c") * NS + lax.axis_index("s")
        pk_bufs = (pk0_v, pk1_v)
        pltpu.sync_copy(bias_hbm, bias_v)
        bias_vec = bias_v[...]
        lane = lax.iota(jnp.int32, LANES)
        nnz0 = wid * (PIX_PER_W * N_ANG)

        def start(buf, c):
            base = nnz0 + (c & (NCHUNK - 1)) * CHUNK_NNZ
            pltpu.async_copy(pk_hbm.at[pl.ds(base, CHUNK_NNZ)],
                             pk_bufs[buf], sems.at[buf])

        def wait(buf):
            pltpu.make_async_copy(pk_hbm.at[pl.ds(0, CHUNK_NNZ)],
                                  pk_bufs[buf], sems.at[buf]).wait()

        def compute(buf, c):
            pkb = pk_bufs[buf]
            for pb in range(CHUNK_PIX // LANES):
                idx0 = lane * N_ANG + pb * (LANES * N_ANG)

                def k_body(k, carry):
                    acc0, acc1, idxv = carry
                    for u in range(UNROLL):
                        w = plsc.load_gather(pkb, [idxv + u])
                        row = w & ROW_MASK
                        vb = (w >> 17) & 0x7FFF
                        vt = plsc.bitcast((vb + VAL_BIAS) << 13, jnp.float32)
                        tg = plsc.load_gather(table_v, [row])
                        lo = plsc.bitcast(tg << 16, jnp.float32)
                        hi = plsc.bitcast(tg & HI_MASK, jnp.float32)
                        acc0 = acc0 + lo * vt
                        acc1 = acc1 + hi * vt
                    return acc0, acc1, idxv + UNROLL

                acc0, acc1, _ = lax.fori_loop(
                    0, N_ANG // UNROLL, k_body,
                    (jnp.zeros((LANES,), jnp.float32),
                     jnp.zeros((LANES,), jnp.float32), idx0))
                off = c * CHUNK_PIX + pb * LANES
                out0_v[pl.ds(off, LANES)] = acc0 + bias_vec
                out1_v[pl.ds(off, LANES)] = acc1 + bias_vec

        start(0, 0)
        start(1, 1)
        for p in range(2):
            pltpu.sync_copy(ptab_hbm.at[p], table_v)

            def pair_body(i, _):
                c0 = i * 2
                wait(0)
                compute(0, c0)
                start(0, c0 + 2)
                wait(1)
                compute(1, c0 + 1)
                start(1, c0 + 3)
                return 0

            lax.fori_loop(0, NCHUNK // 2, pair_body, 0)
            obase = wid * PIX_PER_W
            pltpu.sync_copy(out0_v, out_hbm.at[2 * p, pl.ds(obase, PIX_PER_W)])
            pltpu.sync_copy(out1_v,
                            out_hbm.at[2 * p + 1, pl.ds(obase, PIX_PER_W)])
        wait(0)
        wait(1)

    return backproject


def kernel(sin_fan, fbp_filter, A_vals, scale, bias, A_rows, A_cols):
    x = sin_fan.reshape(B, N_ANG, N_DET)
    xp = jnp.pad(x, ((0, 0), (0, 0), (HALF, HALF)))
    f = fbp_filter.reshape(FILT_LEN) * scale[0]
    c_ix = jnp.arange(PADDED, dtype=jnp.int32)[:, None]
    d_ix = jnp.arange(N_DET, dtype=jnp.int32)[None, :]
    diff = c_ix - d_ix
    band = jnp.where((diff >= 0) & (diff < FILT_LEN),
                     f[jnp.clip(diff, 0, FILT_LEN - 1)], 0.0)
    p01, p23 = _conv_pack(xp, band)
    ptab = jnp.stack([p01.reshape(SIN_SZ), p23.reshape(SIN_SZ)])

    pk = _pack_nnz(A_rows.reshape(PK_ROWS, PK_COLS),
                   A_vals.reshape(PK_ROWS, PK_COLS)).reshape(NNZ)

    bias16 = jnp.broadcast_to(bias.astype(jnp.float32), (LANES,))
    out = _make_backproject()(ptab, pk, bias16)
    return out.reshape(B, OUT_SZ, OUT_SZ, 1)


# speedup vs baseline: 6.7859x; 6.7859x over previous
import functools

import jax
import jax.numpy as jnp
from jax import lax
from jax.experimental import pallas as pl
from jax.experimental.pallas import tpu as pltpu
from jax.experimental.pallas import tpu_sc as plsc

B = 4
N_ANG = 180
N_DET = 384
SIN_SZ = N_ANG * N_DET
OUT_SZ = 256
NPIX = OUT_SZ * OUT_SZ
FILT_LEN = 129
HALF = (FILT_LEN - 1) // 2
PADDED = N_DET + 2 * HALF
NNZ = NPIX * N_ANG

NC = 2
NS = 16
NW = NC * NS
LANES = 16

PIX_PER_W = NPIX // NW
CHUNK_PIX = 128
NCHUNK = PIX_PER_W // CHUNK_PIX
CHUNK_NNZ = CHUNK_PIX * N_ANG
UNROLL = 6

ROW_MASK = (1 << 17) - 1
VAL_BIAS = 114688
HI_MASK = jnp.int32(-65536)


def _conv_pack_body(xp_ref, kt_ref, ptab_ref):
    kt = kt_ref[...]
    y = [lax.dot_general(xp_ref[b], kt, (((1,), (1,)), ((), ())),
                         preferred_element_type=jnp.float32)
         for b in range(B)]
    r = [lax.bitcast_convert_type(v, jnp.uint32) + jnp.uint32(0x8000)
         for v in y]
    ptab_ref[0] = lax.bitcast_convert_type(
        (r[0] >> 16) | (r[1] & jnp.uint32(0xFFFF0000)), jnp.int32)
    ptab_ref[1] = lax.bitcast_convert_type(
        (r[2] >> 16) | (r[3] & jnp.uint32(0xFFFF0000)), jnp.int32)


_conv_pack = pl.pallas_call(
    _conv_pack_body,
    out_shape=jax.ShapeDtypeStruct((2, N_ANG, N_DET), jnp.int32),
)

PK_BLK = NNZ // 16


def _pack_nnz_body(rows_ref, vals_ref, pk_ref):
    vbits = lax.bitcast_convert_type(vals_ref[...], jnp.uint32)
    vb = ((vbits + jnp.uint32(0x1000)) >> 13).astype(jnp.int32) - VAL_BIAS
    vb = jnp.clip(vb, 0, (1 << 15) - 1)
    pk_ref[...] = rows_ref[...] | (vb << 17)


_pack_nnz = pl.pallas_call(
    _pack_nnz_body,
    grid=(NNZ // PK_BLK,),
    in_specs=[pl.BlockSpec((PK_BLK,), lambda i: (i,)),
              pl.BlockSpec((PK_BLK,), lambda i: (i,))],
    out_specs=pl.BlockSpec((PK_BLK,), lambda i: (i,)),
    out_shape=jax.ShapeDtypeStruct((NNZ,), jnp.int32),
)


@functools.lru_cache(maxsize=1)
def _make_backproject():
    mesh = plsc.VectorSubcoreMesh(
        core_axis_name="c", subcore_axis_name="s",
        num_cores=NC, num_subcores=NS)

    @functools.partial(
        pl.kernel,
        mesh=mesh,
        compiler_params=pltpu.CompilerParams(needs_layout_passes=False),
        out_type=jax.ShapeDtypeStruct((B, NPIX), jnp.float32),
        scratch_types=[
            pltpu.VMEM((SIN_SZ,), jnp.int32),
            pltpu.VMEM((CHUNK_NNZ,), jnp.int32),
            pltpu.VMEM((CHUNK_NNZ,), jnp.int32),
            pltpu.VMEM((PIX_PER_W,), jnp.float32),
            pltpu.VMEM((PIX_PER_W,), jnp.float32),
            pltpu.VMEM((LANES,), jnp.float32),
            pltpu.SemaphoreType.DMA((2,)),
        ],
    )
    def backproject(ptab_hbm, pk_hbm, bias_hbm, out_hbm,
                    table_v, pk0_v, pk1_v, out0_v, out1_v, bias_v, sems):
        wid = lax.axis_index("c") * NS + lax.axis_index("s")
        pk_bufs = (pk0_v, pk1_v)
        pltpu.sync_copy(bias_hbm, bias_v)
        bias_vec = bias_v[...]
        lane = lax.iota(jnp.int32, LANES)
        nnz0 = wid * (PIX_PER_W * N_ANG)

        def start(buf, c):
            base = nnz0 + (c & (NCHUNK - 1)) * CHUNK_NNZ
            pltpu.async_copy(pk_hbm.at[pl.ds(base, CHUNK_NNZ)],
                             pk_bufs[buf], sems.at[buf])

        def wait(buf):
            pltpu.make_async_copy(pk_hbm.at[pl.ds(0, CHUNK_NNZ)],
                                  pk_bufs[buf], sems.at[buf]).wait()

        def compute(buf, c):
            pkb = pk_bufs[buf]
            for pb in range(CHUNK_PIX // LANES):
                idx0 = lane * N_ANG + pb * (LANES * N_ANG)

                def k_body(k, carry):
                    acc0, acc1, idxv = carry
                    for u in range(UNROLL):
                        w = plsc.load_gather(pkb, [idxv + u])
                        row = w & ROW_MASK
                        vb = (w >> 17) & 0x7FFF
                        vt = plsc.bitcast((vb + VAL_BIAS) << 13, jnp.float32)
                        tg = plsc.load_gather(table_v, [row])
                        lo = plsc.bitcast(tg << 16, jnp.float32)
                        hi = plsc.bitcast(tg & HI_MASK, jnp.float32)
                        acc0 = acc0 + lo * vt
                        acc1 = acc1 + hi * vt
                    return acc0, acc1, idxv + UNROLL

                acc0, acc1, _ = lax.fori_loop(
                    0, N_ANG // UNROLL, k_body,
                    (jnp.zeros((LANES,), jnp.float32),
                     jnp.zeros((LANES,), jnp.float32), idx0))
                off = c * CHUNK_PIX + pb * LANES
                out0_v[pl.ds(off, LANES)] = acc0 + bias_vec
                out1_v[pl.ds(off, LANES)] = acc1 + bias_vec

        start(0, 0)
        start(1, 1)
        for p in range(2):
            pltpu.sync_copy(ptab_hbm.at[p], table_v)

            def pair_body(i, _):
                c0 = i * 2
                wait(0)
                compute(0, c0)
                start(0, c0 + 2)
                wait(1)
                compute(1, c0 + 1)
                start(1, c0 + 3)
                return 0

            lax.fori_loop(0, NCHUNK // 2, pair_body, 0)
            obase = wid * PIX_PER_W
            pltpu.sync_copy(out0_v, out_hbm.at[2 * p, pl.ds(obase, PIX_PER_W)])
            pltpu.sync_copy(out1_v,
                            out_hbm.at[2 * p + 1, pl.ds(obase, PIX_PER_W)])
        wait(0)
        wait(1)

    return backproject


def kernel(sin_fan, fbp_filter, A_vals, scale, bias, A_rows, A_cols):
    x = sin_fan.reshape(B, N_ANG, N_DET)
    xp = jnp.pad(x, ((0, 0), (0, 0), (HALF, HALF)))
    f = fbp_filter.reshape(FILT_LEN) * scale[0]
    h = jnp.concatenate([f, jnp.zeros((N_DET,), jnp.float32)])
    kt = jnp.tile(h, N_DET)[: N_DET * PADDED].reshape(N_DET, PADDED)
    ptab = _conv_pack(xp, kt).reshape(2, SIN_SZ)

    pk = _pack_nnz(A_rows, A_vals)

    bias16 = jnp.broadcast_to(bias.astype(jnp.float32), (LANES,))
    out = _make_backproject()(ptab, pk, bias16)
    return out.reshape(B, OUT_SZ, OUT_SZ, 1)
